# baseline (speedup 1.0000x reference)
import jax
import jax.numpy as jnp
from jax.experimental import pallas as pl
from jax.experimental.pallas import tpu as pltpu

_TOPK = 8
_KC = 4
_LROWS = 8
_NEG = -3.0e38


def _body(emb_ref, mask_ref, w_ref, b_ref, pooled_ref, attn_ref,
          scores_s, rows_s, cand_s):
    c = pl.program_id(1)
    e = emb_ref[0, 0]
    CH, D = e.shape
    C = CH // _LROWS
    s = jax.lax.dot_general(
        e, w_ref[...],
        dimension_numbers=(((1,), (0,)), ((), ())),
        precision=jax.lax.Precision.DEFAULT,
        preferred_element_type=jnp.float32,
    )
    s = s.reshape(_LROWS, C)
    m = mask_ref[0, 0]
    s = jnp.where(m == 0.0, -jnp.inf, s)

    row_i = jax.lax.broadcasted_iota(jnp.int32, (_LROWS, C), 0)
    col_i = jax.lax.broadcasted_iota(jnp.int32, (_LROWS, C), 1)
    lgrid = row_i * C + col_i
    scores_s[pl.ds(c * _LROWS, _LROWS), :] = s

    s_work = jnp.maximum(s, _NEG)
    lane32 = jax.lax.broadcasted_iota(jnp.int32, (1, _KC * _TOPK), 1)
    fresh = jnp.full((1, _KC * _TOPK), -1, dtype=jnp.int32)
    cv = jnp.where(c == 0, fresh, cand_s[...])
    for j in range(_TOPK):
        v = jnp.max(s_work)
        lidx = jnp.min(jnp.where(s_work == v, lgrid, CH))
        sel = lgrid == lidx
        s_work = jnp.where(sel, -jnp.inf, s_work)
        slot = c * _TOPK + j
        rows_s[pl.ds(slot, 1), :] = emb_ref[0, 0, pl.ds(lidx, 1), :]
        cv = jnp.where(lane32 == slot, c * CH + lidx, cv)
    cand_s[...] = cv

    @pl.when(c == _KC - 1)
    def _():
        K = _KC * CH
        S = scores_s[...]
        rr = jax.lax.broadcasted_iota(jnp.int32, S.shape, 0)
        cc = jax.lax.broadcasted_iota(jnp.int32, S.shape, 1)
        gg = rr * C + cc
        sw = jnp.maximum(S, _NEG)
        cvf = cand_s[...]
        slot_i = jax.lax.broadcasted_iota(jnp.int32, cvf.shape, 1)
        attn = jnp.zeros(S.shape, dtype=jnp.float32)
        pooled = jnp.zeros((1, D), dtype=jnp.float32)
        inv_k = jnp.float32(1.0 / _TOPK)
        for j in range(_TOPK):
            v = jnp.max(sw)
            gidx = jnp.min(jnp.where(sw == v, gg, K))
            sel = gg == gidx
            attn = attn + jnp.where(sel, inv_k, 0.0)
            sw = jnp.where(sel, -jnp.inf, sw)
            slot = jnp.min(jnp.where(cvf == gidx, slot_i, _KC * _TOPK))
            pooled = pooled + rows_s[pl.ds(slot, 1), :] * inv_k
        pooled_ref[0] = pooled
        attn_ref[0] = attn


def kernel(embeddings, mask, W, b):
    B, K, D = embeddings.shape
    CH = K // _KC
    C = CH // _LROWS
    w_t = W.reshape(D, 1)
    b2 = b.reshape(1, 1)
    e5 = embeddings.reshape(B, _KC, CH, D)
    mask5 = mask.reshape(B, _KC, _LROWS, C)
    pooled, attn = pl.pallas_call(
        _body,
        grid=(B, _KC),
        in_specs=[
            pl.BlockSpec((1, 1, CH, D), lambda i, c: (i, c, 0, 0)),
            pl.BlockSpec((1, 1, _LROWS, C), lambda i, c: (i, c, 0, 0)),
            pl.BlockSpec((D, 1), lambda i, c: (0, 0)),
            pl.BlockSpec((1, 1), lambda i, c: (0, 0)),
        ],
        out_specs=[
            pl.BlockSpec((1, 1, D), lambda i, c: (i, 0, 0)),
            pl.BlockSpec((1, _KC * _LROWS, C), lambda i, c: (i, 0, 0)),
        ],
        out_shape=[
            jax.ShapeDtypeStruct((B, 1, D), jnp.float32),
            jax.ShapeDtypeStruct((B, _KC * _LROWS, C), jnp.float32),
        ],
        scratch_shapes=[
            pltpu.VMEM((_KC * _LROWS, C), jnp.float32),
            pltpu.VMEM((_KC * _TOPK, D), jnp.float32),
            pltpu.VMEM((1, _KC * _TOPK), jnp.int32),
        ],
    )(e5, mask5, w_t, b2)
    return (pooled.reshape(B, D), attn.reshape(B, K))


# speedup vs baseline: 3.2694x; 3.2694x over previous
import jax
import jax.numpy as jnp
from jax.experimental import pallas as pl

_TOPK = 8
_ROWS = 32


def _fused_body(emb_ref, mask_ref, w_ref, b_ref, pooled_ref, attn_ref):
    e = emb_ref[0]
    K, D = e.shape
    C = K // _ROWS
    s = jax.lax.dot_general(
        e, w_ref[...],
        dimension_numbers=(((1,), (0,)), ((), ())),
        precision=jax.lax.Precision.DEFAULT,
        preferred_element_type=jnp.float32,
    )
    s = s.reshape(_ROWS, C) + b_ref[0, 0]
    m = mask_ref[0]
    s = jnp.where(m == 0.0, -jnp.inf, s)

    row_i = jax.lax.broadcasted_iota(jnp.int32, (_ROWS, C), 0)
    col_i = jax.lax.broadcasted_iota(jnp.int32, (_ROWS, C), 1)
    gidx = row_i * C + col_i
    s_work = jnp.maximum(s, jnp.float32(-3.0e38))
    attn = jnp.zeros((_ROWS, C), dtype=jnp.float32)
    pooled = jnp.zeros((1, D), dtype=jnp.float32)
    inv_k = jnp.float32(1.0 / _TOPK)
    for _ in range(_TOPK):
        v = jnp.max(s_work)
        cand = jnp.where(s_work == v, gidx, K)
        idx = jnp.min(cand)
        sel = gidx == idx
        attn = attn + jnp.where(sel, inv_k, 0.0)
        s_work = jnp.where(sel, -jnp.inf, s_work)
        row = emb_ref[0, pl.ds(idx, 1), :]
        pooled = pooled + row * inv_k
    pooled_ref[0] = pooled
    attn_ref[0] = attn


def kernel(embeddings, mask, W, b):
    B, K, D = embeddings.shape
    C = K // _ROWS
    b2 = b.reshape(1, 1)
    w_t = W.reshape(D, 1)
    mask4 = mask.reshape(B, _ROWS, C)
    pooled, attn = pl.pallas_call(
        _fused_body,
        grid=(B,),
        in_specs=[
            pl.BlockSpec((1, K, D), lambda i: (i, 0, 0)),
            pl.BlockSpec((1, _ROWS, C), lambda i: (i, 0, 0)),
            pl.BlockSpec((D, 1), lambda i: (0, 0)),
            pl.BlockSpec((1, 1), lambda i: (0, 0)),
        ],
        out_specs=[
            pl.BlockSpec((1, 1, D), lambda i: (i, 0, 0)),
            pl.BlockSpec((1, _ROWS, C), lambda i: (i, 0, 0)),
        ],
        out_shape=[
            jax.ShapeDtypeStruct((B, 1, D), jnp.float32),
            jax.ShapeDtypeStruct((B, _ROWS, C), jnp.float32),
        ],
    )(embeddings, mask4, w_t, b2)
    return (pooled.reshape(B, D), attn.reshape(B, K))
